# baseline (speedup 1.0000x reference)
import functools

import jax
import jax.numpy as jnp
from jax import lax
from jax.experimental import pallas as pl
from jax.experimental.pallas import tpu as pltpu
from jax.experimental.pallas import tpu_sc as plsc

VOCAB = 1000001
EMB = 64
SEQ = 200
BATCH = 4096
NUM_CORES = 2
NUM_SUBCORES = 16
NW = NUM_CORES * NUM_SUBCORES
BPW = BATCH // NW
LANES = 16
BC = 16384
NBLK = 62
VPAD = NBLK * BC
HALF = VPAD // 2


def _tc_tw(table_t, w_row, b2):
    def body(t0_ref, t1_ref, w_ref, b_ref, o0_ref, o1_ref):
        tb = b_ref[...] * (1.0 / SEQ)
        o0_ref[...] = (
            lax.dot_general(w_ref[...], t0_ref[...], (((1,), (0,)), ((), ())),
                            preferred_element_type=jnp.float32)
            * (1.0 / SEQ) + tb
        )
        o1_ref[...] = (
            lax.dot_general(w_ref[...], t1_ref[...], (((1,), (0,)), ((), ())),
                            preferred_element_type=jnp.float32)
            * (1.0 / SEQ) + tb
        )

    return pl.pallas_call(
        body,
        grid=(NBLK // 2,),
        in_specs=[
            pl.BlockSpec((EMB, BC), lambda i: (0, i)),
            pl.BlockSpec((EMB, BC), lambda i: (0, i + NBLK // 2)),
            pl.BlockSpec((1, EMB), lambda i: (0, 0)),
            pl.BlockSpec((1, 1), lambda i: (0, 0)),
        ],
        out_specs=[
            pl.BlockSpec((1, BC), lambda i: (0, i)),
            pl.BlockSpec((1, BC), lambda i: (0, i)),
        ],
        out_shape=[
            jax.ShapeDtypeStruct((1, HALF), jnp.float32),
            jax.ShapeDtypeStruct((1, HALF), jnp.float32),
        ],
    )(table_t, table_t, w_row, b2)


G = 512
NG = SEQ * BPW // G
NRING = 5


def _sc_gather_sum(tw_lo, tw_hi, x4):
    mesh = plsc.VectorSubcoreMesh(
        core_axis_name="c", subcore_axis_name="s",
        num_cores=NUM_CORES, num_subcores=NUM_SUBCORES)

    @functools.partial(
        pl.kernel,
        out_type=jax.ShapeDtypeStruct((BATCH,), jnp.float32),
        mesh=mesh,
        scratch_types=[
            pltpu.VMEM((NG, G), jnp.int32),
            pltpu.VMEM((BPW,), jnp.float32),
            pltpu.VMEM((NRING, G), jnp.float32),
            pltpu.VMEM_SHARED((VPAD,), jnp.float32),
            [pltpu.SemaphoreType.DMA] * NRING,
        ],
        compiler_params=pltpu.CompilerParams(use_tc_tiling_on_sc=False),
    )
    def k(lo_hbm, hi_hbm, x_hbm, out_hbm, idx_v, acc_v, buf_v, tw_sp, sems):
        wid = lax.axis_index("s") * NUM_CORES + lax.axis_index("c")
        sid = lax.axis_index("s")
        base = wid * BPW

        twc = HALF // NUM_SUBCORES
        st0 = (lo_hbm.at[pl.ds(sid * twc, twc)],
               tw_sp.at[pl.ds(sid * twc, twc)])
        st1 = (hi_hbm.at[pl.ds(sid * twc, twc)],
               tw_sp.at[pl.ds(HALF + sid * twc, twc)])
        st2 = (x_hbm.at[wid], idx_v)
        pltpu.async_copy(*st0, sems[0])
        pltpu.async_copy(*st1, sems[1])
        pltpu.async_copy(*st2, sems[2])
        pltpu.make_async_copy(*st0, sems[0]).wait()
        pltpu.make_async_copy(*st1, sems[1]).wait()
        pltpu.make_async_copy(*st2, sems[2]).wait()
        plsc.subcore_barrier()

        for cc in range(BPW // LANES):
            acc_v[pl.ds(cc * LANES, LANES)] = jnp.zeros((LANES,), jnp.float32)

        for u in range(NRING):
            pltpu.async_copy(tw_sp.at[idx_v.at[u]], buf_v.at[u], sems[u])

        def wait_buf(u):
            pltpu.make_async_copy(
                tw_sp.at[idx_v.at[0]], buf_v.at[u], sems[u]).wait()

        def accumulate(u):
            for cc in range(G // LANES):
                sl = pl.ds(cc * LANES, LANES)
                plsc.addupdate(acc_v.at[pl.ds((cc % 8) * LANES, LANES)],
                               buf_v[u, sl])

        def body(kk, carry):
            for u in range(NRING):
                j = NRING * kk + u
                wait_buf(u)
                accumulate(u)

                @pl.when(j + NRING <= NG - 1)
                def _():
                    pltpu.async_copy(
                        tw_sp.at[idx_v.at[j + NRING]], buf_v.at[u], sems[u])
            return carry

        lax.fori_loop(0, NG // NRING - 1, body, None)
        for u in range(NRING):
            wait_buf(u)
            accumulate(u)

        pltpu.sync_copy(acc_v, out_hbm.at[pl.ds(base, BPW)])

    return k(tw_lo, tw_hi, x4)


def kernel(x, table, W, b):
    x32 = x.astype(jnp.int32)
    x4 = x32.reshape(SEQ, NW, BPW).transpose(1, 0, 2).reshape(NW, NG, G)
    tw_lo, tw_hi = _tc_tw(table.T, W.reshape(1, EMB), b.reshape(1, 1))
    return _sc_gather_sum(tw_lo.reshape(HALF), tw_hi.reshape(HALF), x4)


# speedup vs baseline: 1.0044x; 1.0044x over previous
import functools

import jax
import jax.numpy as jnp
from jax import lax
from jax.experimental import pallas as pl
from jax.experimental.pallas import tpu as pltpu
from jax.experimental.pallas import tpu_sc as plsc

VOCAB = 1000001
EMB = 64
SEQ = 200
BATCH = 4096
NUM_CORES = 2
NUM_SUBCORES = 16
NW = NUM_CORES * NUM_SUBCORES
BPW = BATCH // NW
LANES = 16
BC = 16384
NBLK = 62
VPAD = NBLK * BC
HALF = VPAD // 2


def _tc_tw(table_t, w_row, b2):
    def body(t0_ref, t1_ref, w_ref, b_ref, o0_ref, o1_ref):
        tb = b_ref[...] * (1.0 / SEQ)
        o0_ref[...] = (
            lax.dot_general(w_ref[...], t0_ref[...], (((1,), (0,)), ((), ())),
                            preferred_element_type=jnp.float32)
            * (1.0 / SEQ) + tb
        )
        o1_ref[...] = (
            lax.dot_general(w_ref[...], t1_ref[...], (((1,), (0,)), ((), ())),
                            preferred_element_type=jnp.float32)
            * (1.0 / SEQ) + tb
        )

    return pl.pallas_call(
        body,
        grid=(NBLK // 2,),
        in_specs=[
            pl.BlockSpec((EMB, BC), lambda i: (0, i)),
            pl.BlockSpec((EMB, BC), lambda i: (0, i + NBLK // 2)),
            pl.BlockSpec((1, EMB), lambda i: (0, 0)),
            pl.BlockSpec((1, 1), lambda i: (0, 0)),
        ],
        out_specs=[
            pl.BlockSpec((1, BC), lambda i: (0, i)),
            pl.BlockSpec((1, BC), lambda i: (0, i)),
        ],
        out_shape=[
            jax.ShapeDtypeStruct((1, HALF), jnp.float32),
            jax.ShapeDtypeStruct((1, HALF), jnp.float32),
        ],
    )(table_t, table_t, w_row, b2)


G = 512
NG = SEQ * BPW // G
NRING = 5


def _sc_gather_sum(tw_lo, tw_hi, x4):
    mesh = plsc.VectorSubcoreMesh(
        core_axis_name="c", subcore_axis_name="s",
        num_cores=NUM_CORES, num_subcores=NUM_SUBCORES)

    @functools.partial(
        pl.kernel,
        out_type=jax.ShapeDtypeStruct((BATCH,), jnp.float32),
        mesh=mesh,
        scratch_types=[
            pltpu.VMEM((NG, G), jnp.int32),
            pltpu.VMEM((BPW,), jnp.float32),
            pltpu.VMEM((NRING, G), jnp.float32),
            pltpu.VMEM_SHARED((VPAD,), jnp.float32),
            [pltpu.SemaphoreType.DMA] * NRING,
        ],
        compiler_params=pltpu.CompilerParams(use_tc_tiling_on_sc=False),
    )
    def k(lo_hbm, hi_hbm, x_hbm, out_hbm, idx_v, acc_v, buf_v, tw_sp, sems):
        wid = lax.axis_index("s") * NUM_CORES + lax.axis_index("c")
        sid = lax.axis_index("s")
        base = wid * BPW

        twc = HALF // NUM_SUBCORES
        st0 = (lo_hbm.at[pl.ds(sid * twc, twc)],
               tw_sp.at[pl.ds(sid * twc, twc)])
        st1 = (hi_hbm.at[pl.ds(sid * twc, twc)],
               tw_sp.at[pl.ds(HALF + sid * twc, twc)])
        st2 = (x_hbm.at[wid], idx_v)
        pltpu.async_copy(*st0, sems[0])
        pltpu.async_copy(*st1, sems[1])
        pltpu.async_copy(*st2, sems[2])
        pltpu.make_async_copy(*st0, sems[0]).wait()
        pltpu.make_async_copy(*st1, sems[1]).wait()
        pltpu.make_async_copy(*st2, sems[2]).wait()
        plsc.subcore_barrier()

        for cc in range(BPW // LANES):
            acc_v[pl.ds(cc * LANES, LANES)] = jnp.zeros((LANES,), jnp.float32)

        def idx_grp(j):
            return idx_v.at[j]

        for u in range(NRING):
            pltpu.async_copy(tw_sp.at[idx_grp(u)], buf_v.at[u], sems[u])

        def wait_buf(u):
            pltpu.make_async_copy(
                tw_sp.at[idx_grp(0)], buf_v.at[u], sems[u]).wait()

        def accumulate(u):
            for cc in range(G // LANES):
                sl = pl.ds(cc * LANES, LANES)
                plsc.addupdate(acc_v.at[pl.ds((cc % 8) * LANES, LANES)],
                               buf_v[u, sl])

        def body(kk, carry):
            for u in range(NRING):
                j = NRING * kk + u
                wait_buf(u)
                accumulate(u)

                @pl.when(j + NRING <= NG - 1)
                def _():
                    pltpu.async_copy(
                        tw_sp.at[idx_grp(j + NRING)], buf_v.at[u], sems[u])
            return carry

        lax.fori_loop(0, NG // NRING - 1, body, None)
        for u in range(NRING):
            wait_buf(u)
            accumulate(u)

        pltpu.sync_copy(acc_v, out_hbm.at[pl.ds(base, BPW)])

    return k(tw_lo, tw_hi, x4)


def kernel(x, table, W, b):
    x32 = x.astype(jnp.int32)
    x4 = x32.reshape(SEQ, NW, BPW).transpose(1, 0, 2).reshape(NW, NG, G)
    tw_lo, tw_hi = _tc_tw(table.T, W.reshape(1, EMB), b.reshape(1, 1))
    return _sc_gather_sum(tw_lo.reshape(HALF), tw_hi.reshape(HALF), x4)
